# baseline (speedup 1.0000x reference)
import functools

import jax
import jax.numpy as jnp
from jax import lax
from jax.experimental import pallas as pl
from jax.experimental.pallas import tpu as pltpu
from jax.experimental.pallas import tpu_sc as plsc

_NC = 2
_NS = 16
_NW = _NC * _NS


def _make_sc_broadcast(B, S, H, chunk):
    rows_per_w = S // _NW
    n_chunks = rows_per_w // chunk
    mesh = plsc.VectorSubcoreMesh(core_axis_name="c", subcore_axis_name="s")

    @functools.partial(
        pl.kernel,
        mesh=mesh,
        out_type=jax.ShapeDtypeStruct((B, S, H), jnp.float32),
        scratch_types=[
            pltpu.VMEM((chunk, H), jnp.float32),
            pltpu.SemaphoreType.DMA,
        ],
    )
    def sc_broadcast(tab_hbm, out_hbm, buf, sem):
        wid = lax.axis_index("s") * _NC + lax.axis_index("c")
        base = wid * rows_per_w

        def step(j, carry):
            r0 = base + j * chunk
            pltpu.sync_copy(tab_hbm.at[pl.ds(r0, chunk)], buf)
            copies = [
                pltpu.async_copy(buf, out_hbm.at[b, pl.ds(r0, chunk)], sem)
                for b in range(B)
            ]
            for c in copies:
                c.wait()
            return carry

        lax.fori_loop(0, n_chunks, step, 0)

    return sc_broadcast


def _bcast_body(tab_ref, out_ref):
    out_ref[...] = jnp.broadcast_to(tab_ref[...][None, :, :], out_ref.shape)


def _tc_broadcast(table, B, S, H, blk):
    return pl.pallas_call(
        _bcast_body,
        grid=(S // blk,),
        in_specs=[pl.BlockSpec((blk, H), lambda i: (i, 0))],
        out_specs=pl.BlockSpec((B, blk, H), lambda i: (0, i, 0)),
        out_shape=jax.ShapeDtypeStruct((B, S, H), jnp.float32),
    )(table)


def kernel(inputs, position_embeddings):
    B, S, H = inputs.shape
    table = position_embeddings[:S]
    sc_out = _make_sc_broadcast(B, S, H, chunk=64)(table)
    tc_out = _tc_broadcast(table, B, S, H, blk=512)
    return sc_out, tc_out


# speedup vs baseline: 1.6575x; 1.6575x over previous
import functools

import jax
import jax.numpy as jnp
from jax import lax
from jax.experimental import pallas as pl
from jax.experimental.pallas import tpu as pltpu
from jax.experimental.pallas import tpu_sc as plsc

_NC = 2
_NS = 16
_NW = _NC * _NS


def _make_sc_broadcast(B, S, H, chunk):
    rows_per_w = S // _NW
    n_chunks = rows_per_w // chunk
    assert n_chunks % 2 == 0
    mesh = plsc.VectorSubcoreMesh(core_axis_name="c", subcore_axis_name="s")

    @functools.partial(
        pl.kernel,
        mesh=mesh,
        out_type=jax.ShapeDtypeStruct((B, S, H), jnp.float32),
        scratch_types=[
            pltpu.VMEM((chunk, H), jnp.float32),
            pltpu.VMEM((chunk, H), jnp.float32),
            pltpu.SemaphoreType.DMA,
            pltpu.SemaphoreType.DMA,
            pltpu.SemaphoreType.DMA,
            pltpu.SemaphoreType.DMA,
        ],
    )
    def sc_broadcast(tab_hbm, out_hbm, buf0, buf1, si0, si1, so0, so1):
        wid = lax.axis_index("s") * _NC + lax.axis_index("c")
        base = wid * rows_per_w
        bufs = (buf0, buf1)
        sem_in = (si0, si1)
        sem_out = (so0, so1)

        for p in range(2):
            pltpu.async_copy(
                tab_hbm.at[pl.ds(base + p * chunk, chunk)], bufs[p], sem_in[p]
            )

        @pl.loop(0, n_chunks, step=2)
        def _ring(j):
            for p in range(2):
                r0 = base + (j + p) * chunk
                pltpu.make_async_copy(
                    tab_hbm.at[pl.ds(r0, chunk)], bufs[p], sem_in[p]
                ).wait()
                for b in range(B):
                    pltpu.async_copy(
                        bufs[p], out_hbm.at[b, pl.ds(r0, chunk)], sem_out[p]
                    )
            for p in range(2):
                r0 = base + (j + p) * chunk
                for b in range(B):
                    pltpu.make_async_copy(
                        bufs[p], out_hbm.at[b, pl.ds(r0, chunk)], sem_out[p]
                    ).wait()

                @pl.when(j + p + 2 < n_chunks)
                def _refill(p=p, r0=r0):
                    pltpu.async_copy(
                        tab_hbm.at[pl.ds(r0 + 2 * chunk, chunk)],
                        bufs[p],
                        sem_in[p],
                    )

    return sc_broadcast


def kernel(inputs, position_embeddings):
    B, S, H = inputs.shape
    table = position_embeddings[:S]
    return _make_sc_broadcast(B, S, H, chunk=32)(table)
